# baseline (speedup 1.0000x reference)
import jax
import jax.numpy as jnp
from jax import lax
from jax.experimental import pallas as pl
from jax.experimental.pallas import tpu as pltpu
from jax.experimental.pallas import tpu_sc as plsc

NC = 2
NS = 16
NW = NC * NS

BATCH = 16384
N_FIELDS = 26
EMBED_DIM = 64
TOTAL = BATCH * N_FIELDS
PER_W = TOTAL // NW
CHUNK = 128
GPC = 4
GROUP = CHUNK * GPC
NGROUP = PER_W // GROUP
NCHUNK = PER_W // CHUNK


def _body(idx_hbm, table_hbm, out_hbm, idx_v, rows_v, gsem0, gsem1, osem0, osem1):
    wid = lax.axis_index("s") * NC + lax.axis_index("c")

    pltpu.sync_copy(idx_hbm.at[wid], idx_v)

    gsems = (gsem0, gsem1)
    osems = (osem0, osem1)

    def gather_desc(g, h, j):
        return pltpu.make_async_copy(
            table_hbm.at[idx_v.at[g * GPC + j]],
            rows_v.at[h, j],
            gsems[h],
        )

    def out_desc(g, h):
        return pltpu.make_async_copy(rows_v.at[h], out_hbm.at[wid, g], osems[h])

    def start_gathers(g, h):
        for j in range(GPC):
            gather_desc(g, h, j).start()

    def wait_gathers(g, h):
        for j in range(GPC):
            gather_desc(g, h, j).wait()

    start_gathers(0, 0)

    def loop_body(i, carry):
        g0 = 2 * i

        wait_gathers(g0, 0)
        out_desc(g0, 0).start()

        @pl.when(i >= 1)
        def _():
            out_desc(g0 - 1, 1).wait()

        start_gathers(g0 + 1, 1)

        wait_gathers(g0 + 1, 1)
        out_desc(g0 + 1, 1).start()

        @pl.when(i < NGROUP // 2 - 1)
        def _():
            out_desc(g0, 0).wait()
            start_gathers(g0 + 2, 0)

        return carry

    lax.fori_loop(0, NGROUP // 2, loop_body, 0)

    out_desc(NGROUP - 2, 0).wait()
    out_desc(NGROUP - 1, 1).wait()


@jax.jit
def _run(indices, table):
    idx_r = indices.reshape(NW, NCHUNK, CHUNK)
    fn = pl.kernel(
        _body,
        out_type=jax.ShapeDtypeStruct((NW, NGROUP, GPC, CHUNK, EMBED_DIM),
                                      jnp.float32),
        mesh=plsc.VectorSubcoreMesh(core_axis_name="c", subcore_axis_name="s"),
        compiler_params=pltpu.CompilerParams(use_tc_tiling_on_sc=False),
        scratch_types=[
            pltpu.VMEM((NCHUNK, CHUNK), jnp.int32),
            pltpu.VMEM((2, GPC, CHUNK, EMBED_DIM), jnp.float32),
            pltpu.SemaphoreType.DMA,
            pltpu.SemaphoreType.DMA,
            pltpu.SemaphoreType.DMA,
            pltpu.SemaphoreType.DMA,
        ],
    )
    out = fn(idx_r, table)
    return out.reshape(BATCH, N_FIELDS, EMBED_DIM)


def kernel(indices, table):
    return _run(indices, table)


# speedup vs baseline: 1.0063x; 1.0063x over previous
import jax
import jax.numpy as jnp
from jax import lax
from jax.experimental import pallas as pl
from jax.experimental.pallas import tpu as pltpu
from jax.experimental.pallas import tpu_sc as plsc

NC = 2
NS = 16
NW = NC * NS

BATCH = 16384
N_FIELDS = 26
EMBED_DIM = 64
TOTAL = BATCH * N_FIELDS
PER_W = TOTAL // NW
CHUNK = 128
GPC = 2
GROUP = CHUNK * GPC
NBUF = 4
NGROUP = PER_W // GROUP
NCHUNK = PER_W // CHUNK
NITER = NGROUP // NBUF


def _body(idx_hbm, table_hbm, out_hbm, idx_v, rows_v,
          gsem0, gsem1, gsem2, gsem3, osem0, osem1, osem2, osem3):
    wid = lax.axis_index("s") * NC + lax.axis_index("c")

    pltpu.sync_copy(idx_hbm.at[wid], idx_v)

    gsems = (gsem0, gsem1, gsem2, gsem3)
    osems = (osem0, osem1, osem2, osem3)

    def gather_desc(g, h, j):
        return pltpu.make_async_copy(
            table_hbm.at[idx_v.at[g * GPC + j]],
            rows_v.at[h, j],
            gsems[h],
        )

    def out_desc(g, h):
        return pltpu.make_async_copy(rows_v.at[h], out_hbm.at[wid, g], osems[h])

    def start_gathers(g, h):
        for j in range(GPC):
            gather_desc(g, h, j).start()

    def wait_gathers(g, h):
        for j in range(GPC):
            gather_desc(g, h, j).wait()

    start_gathers(0, 0)
    start_gathers(1, 1)

    def loop_body(i, carry):
        g0 = NBUF * i

        wait_gathers(g0, 0)
        out_desc(g0, 0).start()

        @pl.when(i >= 1)
        def _():
            out_desc(g0 - 2, 2).wait()
        start_gathers(g0 + 2, 2)

        wait_gathers(g0 + 1, 1)
        out_desc(g0 + 1, 1).start()

        @pl.when(i >= 1)
        def _():
            out_desc(g0 - 1, 3).wait()
        start_gathers(g0 + 3, 3)

        wait_gathers(g0 + 2, 2)
        out_desc(g0 + 2, 2).start()

        @pl.when(i < NITER - 1)
        def _():
            out_desc(g0, 0).wait()
            start_gathers(g0 + 4, 0)

        wait_gathers(g0 + 3, 3)
        out_desc(g0 + 3, 3).start()

        @pl.when(i < NITER - 1)
        def _():
            out_desc(g0 + 1, 1).wait()
            start_gathers(g0 + 5, 1)

        return carry

    lax.fori_loop(0, NITER, loop_body, 0)

    out_desc(NGROUP - 4, 0).wait()
    out_desc(NGROUP - 3, 1).wait()
    out_desc(NGROUP - 2, 2).wait()
    out_desc(NGROUP - 1, 3).wait()


@jax.jit
def _run(indices, table):
    idx_r = indices.reshape(NW, NCHUNK, CHUNK)
    fn = pl.kernel(
        _body,
        out_type=jax.ShapeDtypeStruct((NW, NGROUP, GPC, CHUNK, EMBED_DIM),
                                      jnp.float32),
        mesh=plsc.VectorSubcoreMesh(core_axis_name="c", subcore_axis_name="s"),
        compiler_params=pltpu.CompilerParams(use_tc_tiling_on_sc=False),
        scratch_types=[
            pltpu.VMEM((NCHUNK, CHUNK), jnp.int32),
            pltpu.VMEM((NBUF, GPC, CHUNK, EMBED_DIM), jnp.float32),
        ] + [pltpu.SemaphoreType.DMA] * 8,
    )
    out = fn(idx_r, table)
    return out.reshape(BATCH, N_FIELDS, EMBED_DIM)


def kernel(indices, table):
    return _run(indices, table)
